# baseline (speedup 1.0000x reference)
import functools

import jax
import jax.numpy as jnp
from jax import lax
from jax.experimental import pallas as pl
from jax.experimental.pallas import tpu as pltpu
from jax.experimental.pallas import tpu_sc as plsc

KNN = 20
KPAD = 24
EPS = 1e-5
NWORKERS = 32
CP = 128


def _knn_call(xt):
    B, N, C = xt.shape

    def body(xt_ref, idx_ref, d_scr):
        b = pl.program_id(0)
        x = xt_ref[0]
        g = lax.dot_general(x, x, (((1,), (1,)), ((), ())),
                            preferred_element_type=jnp.float32)
        xx = jnp.sum(x * x, axis=1)
        inner = -2.0 * g
        d_scr[...] = (-xx[:, None] - inner) - xx[None, :]
        jiota = lax.broadcasted_iota(jnp.int32, (N, N), 1)
        for it in range(KNN):
            dd = d_scr[...]
            m = jnp.max(dd, axis=1)
            jj = jnp.min(jnp.where(dd == m[:, None], jiota, jnp.int32(2**30)),
                         axis=1)
            idx_ref[0, :, it] = jj + b * N
            d_scr[...] = jnp.where(jiota == jj[:, None], -jnp.inf, dd)
        for it in range(KNN, KPAD):
            idx_ref[0, :, it] = jnp.zeros((N,), jnp.int32)

    return pl.pallas_call(
        body,
        grid=(B,),
        in_specs=[pl.BlockSpec((1, N, C), lambda b: (b, 0, 0))],
        out_specs=pl.BlockSpec((1, N, KPAD), lambda b: (b, 0, 0)),
        out_shape=jax.ShapeDtypeStruct((B, N, KPAD), jnp.int32),
        scratch_shapes=[pltpu.VMEM((N, N), jnp.float32)],
    )(xt)


def _gather_call(tab, idx):
    T, cp = tab.shape
    M = T * KPAD
    W = 128
    per = M // NWORKERS
    steps = per // W
    NB = 4
    idxg = idx.reshape(M // W, W)
    mesh = plsc.VectorSubcoreMesh(core_axis_name="c", subcore_axis_name="s")

    @functools.partial(
        pl.kernel,
        out_type=jax.ShapeDtypeStruct((M, cp), jnp.float32),
        mesh=mesh,
        scratch_types=[
            pltpu.VMEM((steps, W), jnp.int32),
        ] + [pltpu.VMEM((W, cp), jnp.float32)] * NB
          + [pltpu.SemaphoreType.DMA] * (2 * NB))
    def k(tab_hbm, idx_hbm, xg_hbm, idx_v, *bufs_and_sems):
        rows = bufs_and_sems[:NB]
        gsem = bufs_and_sems[NB:2 * NB]
        osem = bufs_and_sems[2 * NB:]
        wid = lax.axis_index("s") * 2 + lax.axis_index("c")
        base = wid * steps

        pltpu.sync_copy(idx_hbm.at[pl.ds(base, steps), :], idx_v)

        @pl.loop(0, steps // NB)
        def _grp(jj):
            s0 = jj * NB
            gs = [
                pltpu.async_copy(tab_hbm.at[idx_v.at[s0 + b]], rows[b],
                                 gsem[b])
                for b in range(NB)
            ]
            os = []
            for b in range(NB):
                gs[b].wait()
                os.append(pltpu.async_copy(
                    rows[b], xg_hbm.at[pl.ds((base + s0 + b) * W, W), :],
                    osem[b]))
            for o in os:
                o.wait()

    return k(tab, idxg)


def _conv_call(xg, xt, wd, wc, g, b):
    TKP, cp = xg.shape
    T = TKP // KPAD
    O = wd.shape[0]
    TB = 128
    G = T // TB

    def body(xg_ref, xt_ref, wd_ref, wc_ref, g_ref, b_ref,
             ymax_ref, scale_ref, shift_ref, acc):
        i = pl.program_id(0)

        @pl.when(i == 0)
        def _():
            acc[...] = jnp.zeros_like(acc)

        xn = xt_ref[...]
        xg3 = xg_ref[...].reshape(TB, KPAD, cp)
        diff = (xg3 - xn[:, None, :]).reshape(TB * KPAD, cp)
        yd = lax.dot_general(diff, wd_ref[...], (((1,), (1,)), ((), ())),
                             preferred_element_type=jnp.float32)
        yc = lax.dot_general(xn, wc_ref[...], (((1,), (1,)), ((), ())),
                             preferred_element_type=jnp.float32)
        y3 = yd.reshape(TB, KPAD, O) + yc[:, None, :]
        yk = y3[:, :KNN, :]
        ymax_ref[...] = jnp.max(yk, axis=1)
        acc[0, :] += jnp.sum(yk, axis=(0, 1))
        acc[1, :] += jnp.sum(yk * yk, axis=(0, 1))

        @pl.when(i == G - 1)
        def _():
            cnt = jnp.float32(T * KNN)
            mean = acc[0, :] / cnt
            var = acc[1, :] / cnt - mean * mean
            sc = g_ref[0, :] * lax.rsqrt(var + EPS)
            scale_ref[0, :] = sc
            shift_ref[0, :] = b_ref[0, :] - mean * sc

    return pl.pallas_call(
        body,
        grid=(G,),
        in_specs=[pl.BlockSpec((TB * KPAD, cp), lambda i: (i, 0)),
                  pl.BlockSpec((TB, cp), lambda i: (i, 0)),
                  pl.BlockSpec((O, cp), lambda i: (0, 0)),
                  pl.BlockSpec((O, cp), lambda i: (0, 0)),
                  pl.BlockSpec((1, O), lambda i: (0, 0)),
                  pl.BlockSpec((1, O), lambda i: (0, 0))],
        out_specs=[pl.BlockSpec((TB, O), lambda i: (i, 0)),
                   pl.BlockSpec((1, O), lambda i: (0, 0)),
                   pl.BlockSpec((1, O), lambda i: (0, 0))],
        out_shape=[jax.ShapeDtypeStruct((T, O), jnp.float32),
                   jax.ShapeDtypeStruct((1, O), jnp.float32),
                   jax.ShapeDtypeStruct((1, O), jnp.float32)],
        scratch_shapes=[pltpu.VMEM((2, O), jnp.float32)],
    )(xg, xt, wd, wc, g, b)


def _norm_call(ymax, scale, shift):
    T, O = ymax.shape
    R = 2048
    G = T // R

    def body(ymax_ref, scale_ref, shift_ref, o_ref):
        z = scale_ref[0, :] * ymax_ref[...] + shift_ref[0, :]
        o_ref[...] = jnp.where(z >= 0, z, 0.2 * z)

    return pl.pallas_call(
        body,
        grid=(G,),
        in_specs=[pl.BlockSpec((R, O), lambda i: (i, 0)),
                  pl.BlockSpec((1, O), lambda i: (0, 0)),
                  pl.BlockSpec((1, O), lambda i: (0, 0))],
        out_specs=pl.BlockSpec((R, O), lambda i: (i, 0)),
        out_shape=jax.ShapeDtypeStruct((T, O), jnp.float32),
    )(ymax, scale, shift)


def _final_matmul_call(cat, w5, g5, b5):
    B, N, F = cat.shape
    C = w5.shape[0]

    def body(cat_ref, w5_ref, g_ref, b_ref, yt_ref, scale_ref, shift_ref,
             acc):
        i = pl.program_id(0)

        @pl.when(i == 0)
        def _():
            acc[...] = jnp.zeros_like(acc)

        yt = lax.dot_general(w5_ref[...], cat_ref[0], (((1,), (1,)), ((), ())),
                             preferred_element_type=jnp.float32)
        yt_ref[0] = yt
        acc[:, 0:1] += jnp.sum(yt, axis=1, keepdims=True)
        acc[:, 1:2] += jnp.sum(yt * yt, axis=1, keepdims=True)

        @pl.when(i == B - 1)
        def _():
            cnt = jnp.float32(B * N)
            mean = acc[:, 0:1] / cnt
            var = acc[:, 1:2] / cnt - mean * mean
            sc = g_ref[...] * lax.rsqrt(var + EPS)
            scale_ref[...] = sc
            shift_ref[...] = b_ref[...] - mean * sc

    return pl.pallas_call(
        body,
        grid=(B,),
        in_specs=[pl.BlockSpec((1, N, F), lambda i: (i, 0, 0)),
                  pl.BlockSpec((C, F), lambda i: (0, 0)),
                  pl.BlockSpec((C, 1), lambda i: (0, 0)),
                  pl.BlockSpec((C, 1), lambda i: (0, 0))],
        out_specs=[pl.BlockSpec((1, C, N), lambda i: (i, 0, 0)),
                   pl.BlockSpec((C, 1), lambda i: (0, 0)),
                   pl.BlockSpec((C, 1), lambda i: (0, 0))],
        out_shape=[jax.ShapeDtypeStruct((B, C, N), jnp.float32),
                   jax.ShapeDtypeStruct((C, 1), jnp.float32),
                   jax.ShapeDtypeStruct((C, 1), jnp.float32)],
        scratch_shapes=[pltpu.VMEM((C, 2), jnp.float32)],
    )(cat, w5, g5, b5)


def _final_norm_call(yt, scale, shift):
    B, C, N = yt.shape

    def body(yt_ref, scale_ref, shift_ref, o_ref):
        z = scale_ref[...] * yt_ref[0] + shift_ref[...]
        o_ref[0] = jnp.where(z >= 0, z, 0.2 * z)

    return pl.pallas_call(
        body,
        grid=(B,),
        in_specs=[pl.BlockSpec((1, C, N), lambda i: (i, 0, 0)),
                  pl.BlockSpec((C, 1), lambda i: (0, 0)),
                  pl.BlockSpec((C, 1), lambda i: (0, 0))],
        out_specs=pl.BlockSpec((1, C, N), lambda i: (i, 0, 0)),
        out_shape=jax.ShapeDtypeStruct((B, C, N), jnp.float32),
    )(yt, scale, shift)


def _layer(xt, W, g, b, cin):
    B, N, C = xt.shape
    O = W.shape[0]
    wd = jnp.pad(W[:, :cin], ((0, 0), (0, CP - cin)))
    wc = jnp.pad(W[:, cin:], ((0, 0), (0, CP - cin)))
    idx = _knn_call(xt)
    T = B * N
    tab = jnp.pad(xt.reshape(T, C), ((0, 0), (0, CP - C)))
    xg = _gather_call(tab, idx.reshape(T, KPAD))
    ymax, scale, shift = _conv_call(xg, tab, wd, wc, g.reshape(1, O),
                                    b.reshape(1, O))
    return _norm_call(ymax, scale, shift).reshape(B, N, O)


def kernel(x, W1, g1, b1, W2, g2, b2, W3, g3, b3, W4, g4, b4, W5, g5, b5):
    B, _, N = x.shape
    xt0 = jnp.pad(jnp.transpose(x, (0, 2, 1)), ((0, 0), (0, 0), (0, 5)))
    x1 = _layer(xt0, W1, g1, b1, 3)
    x2 = _layer(x1, W2, g2, b2, 64)
    x3 = _layer(x2, W3, g3, b3, 64)
    x4 = _layer(x3, W4, g4, b4, 128)
    cat = jnp.concatenate([x1, x2, x3, x4], axis=2)
    C = W5.shape[0]
    yt, scale, shift = _final_matmul_call(cat, W5, g5.reshape(C, 1),
                                          b5.reshape(C, 1))
    return _final_norm_call(yt, scale, shift)


# speedup vs baseline: 4.6423x; 4.6423x over previous
import functools

import jax
import jax.numpy as jnp
from jax import lax
from jax.experimental import pallas as pl
from jax.experimental.pallas import tpu as pltpu
from jax.experimental.pallas import tpu_sc as plsc

KNN = 20
KPAD = 24
EPS = 1e-5
NWORKERS = 32
CP = 128


def _knn_call(xt):
    B, N, C = xt.shape

    def body(xt_ref, idx_ref, d_scr):
        b = pl.program_id(0)
        x = xt_ref[0]
        g = lax.dot_general(x, x, (((1,), (1,)), ((), ())),
                            preferred_element_type=jnp.float32)
        xx = jnp.sum(x * x, axis=1)
        inner = -2.0 * g
        d_scr[...] = (-xx[:, None] - inner) - xx[None, :]
        jiota = lax.broadcasted_iota(jnp.int32, (N, N), 1)
        for it in range(KNN):
            dd = d_scr[...]
            m = jnp.max(dd, axis=1)
            jj = jnp.min(jnp.where(dd == m[:, None], jiota, jnp.int32(2**30)),
                         axis=1)
            idx_ref[0, :, it] = jj + b * N
            d_scr[...] = jnp.where(jiota == jj[:, None], -jnp.inf, dd)
        niota = lax.broadcasted_iota(jnp.int32, (N, 1), 0)[:, 0] + b * N
        for it in range(KNN, KPAD):
            idx_ref[0, :, it] = niota

    return pl.pallas_call(
        body,
        grid=(B,),
        in_specs=[pl.BlockSpec((1, N, C), lambda b: (b, 0, 0))],
        out_specs=pl.BlockSpec((1, N, KPAD), lambda b: (b, 0, 0)),
        out_shape=jax.ShapeDtypeStruct((B, N, KPAD), jnp.int32),
        scratch_shapes=[pltpu.VMEM((N, N), jnp.float32)],
    )(xt)


def _gather_call(tab, idx):
    T, cp = tab.shape
    M = T * KPAD
    W = 128
    per = M // NWORKERS
    steps = per // W
    NB = 4
    idxg = idx.reshape(M // W, W)
    mesh = plsc.VectorSubcoreMesh(core_axis_name="c", subcore_axis_name="s")

    @functools.partial(
        pl.kernel,
        out_type=jax.ShapeDtypeStruct((M, cp), jnp.float32),
        mesh=mesh,
        scratch_types=[
            pltpu.VMEM((steps, W), jnp.int32),
        ] + [pltpu.VMEM((W, cp), jnp.float32)] * NB
          + [pltpu.SemaphoreType.DMA] * (2 * NB))
    def k(tab_hbm, idx_hbm, xg_hbm, idx_v, *bufs_and_sems):
        rows = bufs_and_sems[:NB]
        gsem = bufs_and_sems[NB:2 * NB]
        osem = bufs_and_sems[2 * NB:]
        wid = lax.axis_index("s") * 2 + lax.axis_index("c")
        base = wid * steps

        pltpu.sync_copy(idx_hbm.at[pl.ds(base, steps), :], idx_v)

        @pl.loop(0, steps // NB)
        def _grp(jj):
            s0 = jj * NB
            gs = [
                pltpu.async_copy(tab_hbm.at[idx_v.at[s0 + b]], rows[b],
                                 gsem[b])
                for b in range(NB)
            ]
            os = []
            for b in range(NB):
                gs[b].wait()
                os.append(pltpu.async_copy(
                    rows[b], xg_hbm.at[pl.ds((base + s0 + b) * W, W), :],
                    osem[b]))
            for o in os:
                o.wait()

    return k(tab, idxg)


def _conv_call(xg, xt, wd, wc, g, b):
    TKP, cp = xg.shape
    T = TKP // KPAD
    O = wd.shape[0]
    TB = 128
    G = T // TB

    def body(xg_ref, xt_ref, wd_ref, wc_ref, g_ref, b_ref,
             ymax_ref, scale_ref, shift_ref, acc):
        i = pl.program_id(0)

        @pl.when(i == 0)
        def _():
            acc[...] = jnp.zeros_like(acc)

        xn = xt_ref[...]
        xg3 = xg_ref[...].reshape(TB, KPAD, cp)
        diff = (xg3 - xn[:, None, :]).reshape(TB * KPAD, cp)
        yd = lax.dot_general(diff, wd_ref[...], (((1,), (1,)), ((), ())),
                             preferred_element_type=jnp.float32)
        yc = lax.dot_general(xn, wc_ref[...], (((1,), (1,)), ((), ())),
                             preferred_element_type=jnp.float32)
        y3 = yd.reshape(TB, KPAD, O) + yc[:, None, :]
        yk = y3[:, :KNN, :]
        ymax_ref[...] = jnp.max(yk, axis=1)
        acc[0, :] += jnp.sum(yk, axis=(0, 1))
        acc[1, :] += jnp.sum(yk * yk, axis=(0, 1))

        @pl.when(i == G - 1)
        def _():
            cnt = jnp.float32(T * KNN)
            mean = acc[0, :] / cnt
            var = acc[1, :] / cnt - mean * mean
            sc = g_ref[0, :] * lax.rsqrt(var + EPS)
            scale_ref[0, :] = sc
            shift_ref[0, :] = b_ref[0, :] - mean * sc

    return pl.pallas_call(
        body,
        grid=(G,),
        in_specs=[pl.BlockSpec((TB * KPAD, cp), lambda i: (i, 0)),
                  pl.BlockSpec((TB, cp), lambda i: (i, 0)),
                  pl.BlockSpec((O, cp), lambda i: (0, 0)),
                  pl.BlockSpec((O, cp), lambda i: (0, 0)),
                  pl.BlockSpec((1, O), lambda i: (0, 0)),
                  pl.BlockSpec((1, O), lambda i: (0, 0))],
        out_specs=[pl.BlockSpec((TB, O), lambda i: (i, 0)),
                   pl.BlockSpec((1, O), lambda i: (0, 0)),
                   pl.BlockSpec((1, O), lambda i: (0, 0))],
        out_shape=[jax.ShapeDtypeStruct((T, O), jnp.float32),
                   jax.ShapeDtypeStruct((1, O), jnp.float32),
                   jax.ShapeDtypeStruct((1, O), jnp.float32)],
        scratch_shapes=[pltpu.VMEM((2, O), jnp.float32)],
    )(xg, xt, wd, wc, g, b)


def _norm_call(ymax, scale, shift):
    T, O = ymax.shape
    R = 2048
    G = T // R

    def body(ymax_ref, scale_ref, shift_ref, o_ref):
        z = scale_ref[0, :] * ymax_ref[...] + shift_ref[0, :]
        o_ref[...] = jnp.where(z >= 0, z, 0.2 * z)

    return pl.pallas_call(
        body,
        grid=(G,),
        in_specs=[pl.BlockSpec((R, O), lambda i: (i, 0)),
                  pl.BlockSpec((1, O), lambda i: (0, 0)),
                  pl.BlockSpec((1, O), lambda i: (0, 0))],
        out_specs=pl.BlockSpec((R, O), lambda i: (i, 0)),
        out_shape=jax.ShapeDtypeStruct((T, O), jnp.float32),
    )(ymax, scale, shift)


def _final_matmul_call(cat, w5, g5, b5):
    B, N, F = cat.shape
    C = w5.shape[0]

    def body(cat_ref, w5_ref, g_ref, b_ref, yt_ref, scale_ref, shift_ref,
             acc):
        i = pl.program_id(0)

        @pl.when(i == 0)
        def _():
            acc[...] = jnp.zeros_like(acc)

        yt = lax.dot_general(w5_ref[...], cat_ref[0], (((1,), (1,)), ((), ())),
                             preferred_element_type=jnp.float32)
        yt_ref[0] = yt
        acc[:, 0:1] += jnp.sum(yt, axis=1, keepdims=True)
        acc[:, 1:2] += jnp.sum(yt * yt, axis=1, keepdims=True)

        @pl.when(i == B - 1)
        def _():
            cnt = jnp.float32(B * N)
            mean = acc[:, 0:1] / cnt
            var = acc[:, 1:2] / cnt - mean * mean
            sc = g_ref[...] * lax.rsqrt(var + EPS)
            scale_ref[...] = sc
            shift_ref[...] = b_ref[...] - mean * sc

    return pl.pallas_call(
        body,
        grid=(B,),
        in_specs=[pl.BlockSpec((1, N, F), lambda i: (i, 0, 0)),
                  pl.BlockSpec((C, F), lambda i: (0, 0)),
                  pl.BlockSpec((C, 1), lambda i: (0, 0)),
                  pl.BlockSpec((C, 1), lambda i: (0, 0))],
        out_specs=[pl.BlockSpec((1, C, N), lambda i: (i, 0, 0)),
                   pl.BlockSpec((C, 1), lambda i: (0, 0)),
                   pl.BlockSpec((C, 1), lambda i: (0, 0))],
        out_shape=[jax.ShapeDtypeStruct((B, C, N), jnp.float32),
                   jax.ShapeDtypeStruct((C, 1), jnp.float32),
                   jax.ShapeDtypeStruct((C, 1), jnp.float32)],
        scratch_shapes=[pltpu.VMEM((C, 2), jnp.float32)],
    )(cat, w5, g5, b5)


def _final_norm_call(yt, scale, shift):
    B, C, N = yt.shape

    def body(yt_ref, scale_ref, shift_ref, o_ref):
        z = scale_ref[...] * yt_ref[0] + shift_ref[...]
        o_ref[0] = jnp.where(z >= 0, z, 0.2 * z)

    return pl.pallas_call(
        body,
        grid=(B,),
        in_specs=[pl.BlockSpec((1, C, N), lambda i: (i, 0, 0)),
                  pl.BlockSpec((C, 1), lambda i: (0, 0)),
                  pl.BlockSpec((C, 1), lambda i: (0, 0))],
        out_specs=pl.BlockSpec((1, C, N), lambda i: (i, 0, 0)),
        out_shape=jax.ShapeDtypeStruct((B, C, N), jnp.float32),
    )(yt, scale, shift)


def _layer(xt, W, g, b, cin):
    B, N, C = xt.shape
    O = W.shape[0]
    wd = jnp.pad(W[:, :cin], ((0, 0), (0, CP - cin)))
    wc = jnp.pad(W[:, cin:], ((0, 0), (0, CP - cin)))
    idx = _knn_call(xt)
    T = B * N
    tab = jnp.pad(xt.reshape(T, C), ((0, 0), (0, CP - C)))
    xg = _gather_call(tab, idx.reshape(T, KPAD))
    ymax, scale, shift = _conv_call(xg, tab, wd, wc, g.reshape(1, O),
                                    b.reshape(1, O))
    return _norm_call(ymax, scale, shift).reshape(B, N, O)


def kernel(x, W1, g1, b1, W2, g2, b2, W3, g3, b3, W4, g4, b4, W5, g5, b5):
    B, _, N = x.shape
    xt0 = jnp.pad(jnp.transpose(x, (0, 2, 1)), ((0, 0), (0, 0), (0, 5)))
    x1 = _layer(xt0, W1, g1, b1, 3)
    x2 = _layer(x1, W2, g2, b2, 64)
    x3 = _layer(x2, W3, g3, b3, 64)
    x4 = _layer(x3, W4, g4, b4, 128)
    cat = jnp.concatenate([x1, x2, x3, x4], axis=2)
    C = W5.shape[0]
    yt, scale, shift = _final_matmul_call(cat, W5, g5.reshape(C, 1),
                                          b5.reshape(C, 1))
    return _final_norm_call(yt, scale, shift)
